# Initial kernel scaffold; baseline (speedup 1.0000x reference)
import jax
import jax.numpy as jnp
from jax.experimental import pallas as pl


def kernel(xyz, W_proj, b_proj, W10, b10, W11, b11, W20, b20, W21, b21, Wd1, bd1, Wd2, bd2):
    raise NotImplementedError("write your pallas kernel here")


# speedup vs baseline: 4.2236x; 4.2236x over previous
import functools

import jax
import jax.numpy as jnp
from jax.experimental import pallas as pl
from jax.experimental.pallas import tpu as pltpu
from jax.experimental.pallas import tpu_sc as plsc


def _fps_body(m, chunk, x_ref, y_ref, z_ref, nx_ref, ny_ref, nz_ref):
    X = x_ref[...]
    Y = y_ref[...]
    Z = z_ref[...]
    b, n = X.shape
    iota = jax.lax.broadcasted_iota(jnp.int32, (b, n), 1)
    lane = jax.lax.broadcasted_iota(jnp.int32, (b, chunk), 1)

    dists = jnp.full((b, n), 1e10, jnp.float32)
    lx = X[:, 0:1]
    ly = Y[:, 0:1]
    lz = Z[:, 0:1]

    def step(j, carry):
        dists, lx, ly, lz, ax, ay, az = carry
        d = (X - lx) ** 2 + (Y - ly) ** 2 + (Z - lz) ** 2
        dists = jnp.minimum(dists, d)
        mx = jnp.max(dists, axis=1, keepdims=True)
        cand = jnp.where(dists == mx, iota, n)
        nxt = jnp.min(cand, axis=1, keepdims=True)
        oh = iota == nxt
        lx = jnp.sum(jnp.where(oh, X, 0.0), axis=1, keepdims=True)
        ly = jnp.sum(jnp.where(oh, Y, 0.0), axis=1, keepdims=True)
        lz = jnp.sum(jnp.where(oh, Z, 0.0), axis=1, keepdims=True)
        colm = lane == j
        ax = jnp.where(colm, lx, ax)
        ay = jnp.where(colm, ly, ay)
        az = jnp.where(colm, lz, az)
        return dists, lx, ly, lz, ax, ay, az

    for c in range(m // chunk):
        ax = jnp.zeros((b, chunk), jnp.float32)
        ay = jnp.zeros((b, chunk), jnp.float32)
        az = jnp.zeros((b, chunk), jnp.float32)
        if c == 0:
            ax = jnp.where(lane == 0, lx, ax)
            ay = jnp.where(lane == 0, ly, ay)
            az = jnp.where(lane == 0, lz, az)
            start = 1
        else:
            start = 0
        carry = (dists, lx, ly, lz, ax, ay, az)
        carry = jax.lax.fori_loop(start, chunk, step, carry)
        dists, lx, ly, lz, ax, ay, az = carry
        sl = pl.ds(c * chunk, chunk)
        nx_ref[:, sl] = ax
        ny_ref[:, sl] = ay
        nz_ref[:, sl] = az


def _fps(X, Y, Z, m):
    b, n = X.shape
    chunk = min(m, 128)
    out = jax.ShapeDtypeStruct((b, m), jnp.float32)
    return pl.pallas_call(
        functools.partial(_fps_body, m, chunk),
        out_shape=(out, out, out),
    )(X, Y, Z)


def _knn_body(n, nk, x_ref, y_ref, z_ref, qx_ref, qy_ref, qz_ref, out_ref):
    bi = pl.program_id(0)
    X = x_ref[0]
    Y = y_ref[0]
    Z = z_ref[0]
    qx = qx_ref[0]
    qy = qy_ref[0]
    qz = qz_ref[0]
    bf = jnp.bfloat16
    f32 = jnp.float32
    mm = (qx.astype(bf).astype(f32) * X.astype(bf).astype(f32)
          + qy.astype(bf).astype(f32) * Y.astype(bf).astype(f32)
          + qz.astype(bf).astype(f32) * Z.astype(bf).astype(f32))
    na = qx * qx + qy * qy + qz * qz
    nb = X * X + Y * Y + Z * Z
    D = (na + nb) - 2.0 * mm
    iota = jax.lax.broadcasted_iota(jnp.int32, (8, n), 1)
    kl = jax.lax.broadcasted_iota(jnp.int32, (8, nk), 1)
    acc = jnp.zeros((8, nk), jnp.int32)
    for j in range(nk):
        mn = jnp.min(D, axis=1, keepdims=True)
        cand = jnp.where(D == mn, iota, n)
        am = jnp.min(cand, axis=1, keepdims=True)
        acc = jnp.where(kl == j, am, acc)
        D = jnp.where(iota == am, jnp.inf, D)
    out_ref[...] = (acc + bi * n)[None]


def _knn(X, Y, Z, qx, qy, qz, nk):
    b, n = X.shape
    s = qx.shape[1]
    grid = (b, s // 8)
    X3 = X.reshape(b, 1, n)
    Y3 = Y.reshape(b, 1, n)
    Z3 = Z.reshape(b, 1, n)
    qx3 = qx.reshape(b * s // 8, 8, 1)
    qy3 = qy.reshape(b * s // 8, 8, 1)
    qz3 = qz.reshape(b * s // 8, 8, 1)
    ns = s // 8
    pts_spec = pl.BlockSpec((1, 1, n), lambda bi, qi: (bi, 0, 0))
    q_spec = pl.BlockSpec((1, 8, 1), lambda bi, qi: (bi * ns + qi, 0, 0))
    return pl.pallas_call(
        functools.partial(_knn_body, n, nk),
        grid=grid,
        in_specs=[pts_spec] * 3 + [q_spec] * 3,
        out_specs=pl.BlockSpec((1, 8, nk), lambda bi, qi: (bi, qi, 0)),
        out_shape=jax.ShapeDtypeStruct((b, s, nk), jnp.int32),
        compiler_params=pltpu.CompilerParams(
            dimension_semantics=("parallel", "parallel")),
    )(X3, Y3, Z3, qx3, qy3, qz3)


def _pre1_body(xyz_ref, wproj_ref, bproj_ref, w10_ref, b10_ref, out_ref):
    w10x = w10_ref[0:3, :]
    w10f = w10_ref[3:, :]
    M = w10x + jnp.dot(wproj_ref[...], w10f,
                       preferred_element_type=jnp.float32)
    c = jnp.dot(bproj_ref[...], w10f,
                preferred_element_type=jnp.float32) + b10_ref[...]
    x = xyz_ref[0, :, 0:1]
    y = xyz_ref[0, :, 1:2]
    z = xyz_ref[0, :, 2:3]
    out_ref[0] = x * M[0:1, :] + y * M[1:2, :] + z * M[2:3, :] + c


def _pre1(xyz, W_proj, b_proj2, W10, b10_2):
    b, n, _ = xyz.shape
    C = W10.shape[1]
    return pl.pallas_call(
        _pre1_body,
        grid=(b,),
        in_specs=[
            pl.BlockSpec((1, n, 3), lambda bi: (bi, 0, 0)),
            pl.BlockSpec(W_proj.shape, lambda bi: (0, 0)),
            pl.BlockSpec(b_proj2.shape, lambda bi: (0, 0)),
            pl.BlockSpec(W10.shape, lambda bi: (0, 0)),
            pl.BlockSpec(b10_2.shape, lambda bi: (0, 0)),
        ],
        out_specs=pl.BlockSpec((1, n, C), lambda bi: (bi, 0, 0)),
        out_shape=jax.ShapeDtypeStruct((b, n, C), jnp.float32),
        compiler_params=pltpu.CompilerParams(
            dimension_semantics=("parallel",)),
    )(xyz, W_proj, b_proj2, W10, b10_2)


def _pre2_body(xyz_ref, f_ref, w20_ref, b20_ref, out_ref):
    w20x = w20_ref[0:3, :]
    w20f = w20_ref[3:, :]
    x = xyz_ref[0, :, 0:1]
    y = xyz_ref[0, :, 1:2]
    z = xyz_ref[0, :, 2:3]
    acc = jnp.dot(f_ref[0], w20f, preferred_element_type=jnp.float32)
    out_ref[0] = (acc + x * w20x[0:1, :] + y * w20x[1:2, :]
                  + z * w20x[2:3, :] + b20_ref[...])


def _pre2(xyz1, feats1, W20, b20_2):
    b, s, F = feats1.shape
    C = W20.shape[1]
    return pl.pallas_call(
        _pre2_body,
        grid=(b,),
        in_specs=[
            pl.BlockSpec((1, s, 3), lambda bi: (bi, 0, 0)),
            pl.BlockSpec((1, s, F), lambda bi: (bi, 0, 0)),
            pl.BlockSpec(W20.shape, lambda bi: (0, 0)),
            pl.BlockSpec(b20_2.shape, lambda bi: (0, 0)),
        ],
        out_specs=pl.BlockSpec((1, s, C), lambda bi: (bi, 0, 0)),
        out_shape=jax.ShapeDtypeStruct((b, s, C), jnp.float32),
        compiler_params=pltpu.CompilerParams(
            dimension_semantics=("parallel",)),
    )(xyz1, feats1, W20, b20_2)


def _sc_gather(table, flat_idx, window=128):
    nidx = flat_idx.shape[1]
    vdim = table.shape[1]
    mesh = plsc.VectorSubcoreMesh(core_axis_name="core",
                                  subcore_axis_name="subcore")

    @pl.kernel(out_type=jax.ShapeDtypeStruct((nidx, vdim), table.dtype),
               mesh=mesh)
    def gather_kernel(x_hbm, i_hbm, o_hbm):
        def body(i_vmem, o_vmem):
            pltpu.sync_copy(x_hbm.at[i_vmem.at[0]], o_vmem)

        pltpu.emit_pipeline(
            body,
            grid=(nidx // window,),
            in_specs=[pl.BlockSpec((1, window), lambda i: (0, i))],
            out_specs=[pl.BlockSpec((window, vdim), lambda i: (i, 0))],
            core_axis_name=("core", "subcore"),
            dimension_semantics=(pltpu.PARALLEL,),
        )(i_hbm, o_hbm)

    return gather_kernel(table, flat_idx)


def _mlp_body(bq, nk, g_ref, q_ref, w0_ref, w1_ref, b1_ref, out_ref):
    C = w1_ref.shape[0]
    w0x = w0_ref[0:3, :]
    q = q_ref[0]
    qoff = (q[:, 0:1] * w0x[0:1, :] + q[:, 1:2] * w0x[1:2, :]
            + q[:, 2:3] * w0x[2:3, :])
    G = g_ref[0]
    h0 = jnp.maximum(G.reshape(bq, nk, C) - qoff[:, None, :], 0.0)
    h1 = jnp.dot(h0.reshape(bq * nk, C), w1_ref[...],
                 preferred_element_type=jnp.float32) + b1_ref[...]
    h1 = jnp.maximum(h1, 0.0)
    out_ref[0] = jnp.max(h1.reshape(bq, nk, C), axis=1)


def _mlp(G, new_xyz, W0, W1, b1_2, nk, bq):
    b, s, _ = new_xyz.shape
    C = W1.shape[0]
    grid = (b, s // bq)
    return pl.pallas_call(
        functools.partial(_mlp_body, bq, nk),
        grid=grid,
        in_specs=[
            pl.BlockSpec((1, bq * nk, C), lambda bi, qi: (bi, qi, 0)),
            pl.BlockSpec((1, bq, 3), lambda bi, qi: (bi, qi, 0)),
            pl.BlockSpec(W0.shape, lambda bi, qi: (0, 0)),
            pl.BlockSpec(W1.shape, lambda bi, qi: (0, 0)),
            pl.BlockSpec(b1_2.shape, lambda bi, qi: (0, 0)),
        ],
        out_specs=pl.BlockSpec((1, bq, C), lambda bi, qi: (bi, qi, 0)),
        out_shape=jax.ShapeDtypeStruct((b, s, C), jnp.float32),
        compiler_params=pltpu.CompilerParams(
            dimension_semantics=("parallel", "parallel")),
    )(G, new_xyz, W0, W1, b1_2)


def _mlp2_body(bq, nk, g_ref, q_ref, w0_ref, w1_ref, b1_ref,
               wd1_ref, bd1_ref, wd2_ref, bd2_ref, out_ref):
    C = w1_ref.shape[0]
    w0x = w0_ref[0:3, :]
    q = q_ref[0]
    qoff = (q[:, 0:1] * w0x[0:1, :] + q[:, 1:2] * w0x[1:2, :]
            + q[:, 2:3] * w0x[2:3, :])
    G = g_ref[0]
    h0 = jnp.maximum(G.reshape(bq, nk, C) - qoff[:, None, :], 0.0)
    h1 = jnp.dot(h0.reshape(bq * nk, C), w1_ref[...],
                 preferred_element_type=jnp.float32) + b1_ref[...]
    h1 = jnp.maximum(h1, 0.0)
    f2 = jnp.max(h1.reshape(bq, nk, C), axis=1)
    fmax = jnp.max(f2, axis=0, keepdims=True)
    favg = jnp.sum(f2, axis=0, keepdims=True) * (1.0 / bq)
    gfeat = jnp.concatenate([fmax, favg], axis=1)
    h = jnp.dot(gfeat, wd1_ref[...],
                preferred_element_type=jnp.float32) + bd1_ref[...]
    h = jnp.maximum(h, 0.0)
    o = jnp.dot(h, wd2_ref[...],
                preferred_element_type=jnp.float32) + bd2_ref[...]
    out_ref[0] = jnp.maximum(o, 0.0)


def _mlp2(G, new_xyz, W0, W1, b1_2, Wd1, bd1_2, Wd2, bd2_2, nk):
    b, s, _ = new_xyz.shape
    C = W1.shape[0]
    O = Wd2.shape[1]
    return pl.pallas_call(
        functools.partial(_mlp2_body, s, nk),
        grid=(b,),
        in_specs=[
            pl.BlockSpec((1, s * nk, C), lambda bi: (bi, 0, 0)),
            pl.BlockSpec((1, s, 3), lambda bi: (bi, 0, 0)),
            pl.BlockSpec(W0.shape, lambda bi: (0, 0)),
            pl.BlockSpec(W1.shape, lambda bi: (0, 0)),
            pl.BlockSpec(b1_2.shape, lambda bi: (0, 0)),
            pl.BlockSpec(Wd1.shape, lambda bi: (0, 0)),
            pl.BlockSpec(bd1_2.shape, lambda bi: (0, 0)),
            pl.BlockSpec(Wd2.shape, lambda bi: (0, 0)),
            pl.BlockSpec(bd2_2.shape, lambda bi: (0, 0)),
        ],
        out_specs=pl.BlockSpec((1, 1, O), lambda bi: (bi, 0, 0)),
        out_shape=jax.ShapeDtypeStruct((b, 1, O), jnp.float32),
        compiler_params=pltpu.CompilerParams(
            dimension_semantics=("parallel",)),
    )(G, new_xyz, W0, W1, b1_2, Wd1, bd1_2, Wd2, bd2_2)


def kernel(xyz, W_proj, b_proj, W10, b10, W11, b11, W20, b20, W21, b21,
           Wd1, bd1, Wd2, bd2):
    b, n, _ = xyz.shape
    m1, m2, nk = 1024, 256, 16
    C1 = W11.shape[0]
    C2 = W21.shape[0]

    X = xyz[..., 0]
    Y = xyz[..., 1]
    Z = xyz[..., 2]

    nx1, ny1, nz1 = _fps(X, Y, Z, m1)
    flat1 = _knn(X, Y, Z, nx1, ny1, nz1, nk)
    preA1 = _pre1(xyz, W_proj, b_proj.reshape(1, -1), W10,
                  b10.reshape(1, -1))
    G1 = _sc_gather(preA1.reshape(b * n, C1), flat1.reshape(1, -1))
    new_xyz1 = jnp.stack([nx1, ny1, nz1], axis=-1)
    feats1 = _mlp(G1.reshape(b, m1 * nk, C1), new_xyz1, W10, W11,
                  b11.reshape(1, -1), nk, bq=128)

    nx2, ny2, nz2 = _fps(nx1, ny1, nz1, m2)
    flat2 = _knn(nx1, ny1, nz1, nx2, ny2, nz2, nk)
    preA2 = _pre2(new_xyz1, feats1, W20, b20.reshape(1, -1))
    G2 = _sc_gather(preA2.reshape(b * m1, C2), flat2.reshape(1, -1))
    new_xyz2 = jnp.stack([nx2, ny2, nz2], axis=-1)

    out = _mlp2(G2.reshape(b, m2 * nk, C2), new_xyz2, W20, W21,
                b21.reshape(1, -1), Wd1, bd1.reshape(1, -1), Wd2,
                bd2.reshape(1, -1), nk)
    return out


# speedup vs baseline: 16.3140x; 3.8625x over previous
import functools

import jax
import jax.numpy as jnp
from jax.experimental import pallas as pl
from jax.experimental.pallas import tpu as pltpu
from jax.experimental.pallas import tpu_sc as plsc


def _fps_body(m, chunk, rpb, x_ref, y_ref, z_ref, nx_ref, ny_ref, nz_ref):
    nc, _, w = x_ref.shape
    rp = jax.lax.broadcasted_iota(jnp.int32, (8, 1), 0) % rpb
    lanec = jax.lax.broadcasted_iota(jnp.int32, (8, chunk), 1)
    lane_w = jax.lax.broadcasted_iota(jnp.int32, (8, w), 1)
    iota_flat = lane_w + rp * w

    def gbcast(v):
        stride = 1
        while stride < rpb:
            vr = pltpu.roll(v, stride, axis=0)
            sel = (rp % (2 * stride)) >= stride
            v = jnp.where(sel, vr, v)
            stride *= 2
        return v

    def gmax(mv, fl):
        stride = 1
        while stride < rpb:
            mv2 = pltpu.roll(mv, 8 - stride, axis=0)
            fl2 = pltpu.roll(fl, 8 - stride, axis=0)
            take = mv2 > mv
            mv = jnp.where(take, mv2, mv)
            fl = jnp.where(take, fl2, fl)
            stride *= 2
        return gbcast(fl)

    def gsum(v):
        stride = 1
        while stride < rpb:
            v = v + pltpu.roll(v, 8 - stride, axis=0)
            stride *= 2
        return gbcast(v)

    X = [x_ref[c] for c in range(nc)]
    Y = [y_ref[c] for c in range(nc)]
    Z = [z_ref[c] for c in range(nc)]
    lx = [gbcast(X[c][:, 0:1]) for c in range(nc)]
    ly = [gbcast(Y[c][:, 0:1]) for c in range(nc)]
    lz = [gbcast(Z[c][:, 0:1]) for c in range(nc)]
    dists = [jnp.full((8, w), 1e10, jnp.float32) for _ in range(nc)]

    def step(j, carry):
        ds, lxs, lys, lzs, axs, ays, azs = [list(t) for t in carry]
        for c in range(nc):
            d = ((X[c] - lxs[c]) ** 2 + (Y[c] - lys[c]) ** 2
                 + (Z[c] - lzs[c]) ** 2)
            dd = jnp.minimum(ds[c], d)
            am = jnp.argmax(dd, axis=1, keepdims=True).astype(jnp.int32)
            mv = jnp.max(dd, axis=1, keepdims=True)
            fl = gmax(mv, am + rp * w)
            oh = iota_flat == fl
            nlx = gsum(jnp.sum(jnp.where(oh, X[c], 0.0), axis=1,
                               keepdims=True))
            nly = gsum(jnp.sum(jnp.where(oh, Y[c], 0.0), axis=1,
                               keepdims=True))
            nlz = gsum(jnp.sum(jnp.where(oh, Z[c], 0.0), axis=1,
                               keepdims=True))
            colm = lanec == j
            ds[c] = dd
            lxs[c], lys[c], lzs[c] = nlx, nly, nlz
            axs[c] = jnp.where(colm, nlx, axs[c])
            ays[c] = jnp.where(colm, nly, ays[c])
            azs[c] = jnp.where(colm, nlz, azs[c])
        return (tuple(ds), tuple(lxs), tuple(lys), tuple(lzs),
                tuple(axs), tuple(ays), tuple(azs))

    for ci in range(m // chunk):
        axs = [jnp.zeros((8, chunk), jnp.float32) for _ in range(nc)]
        ays = [jnp.zeros((8, chunk), jnp.float32) for _ in range(nc)]
        azs = [jnp.zeros((8, chunk), jnp.float32) for _ in range(nc)]
        if ci == 0:
            axs = [jnp.where(lanec == 0, lx[c], axs[c]) for c in range(nc)]
            ays = [jnp.where(lanec == 0, ly[c], ays[c]) for c in range(nc)]
            azs = [jnp.where(lanec == 0, lz[c], azs[c]) for c in range(nc)]
            start = 1
        else:
            start = 0
        carry = (tuple(dists), tuple(lx), tuple(ly), tuple(lz),
                 tuple(axs), tuple(ays), tuple(azs))
        carry = jax.lax.fori_loop(start, chunk, step, carry)
        dists, lx, ly, lz, axs, ays, azs = [list(t) for t in carry]
        sl = pl.ds(ci * chunk, chunk)
        for c in range(nc):
            nx_ref[c, :, sl] = axs[c]
            ny_ref[c, :, sl] = ays[c]
            nz_ref[c, :, sl] = azs[c]


def _fps(X, Y, Z, m, nc=4):
    b, n = X.shape
    rpb = 8 * nc // b
    w = n // rpb
    chunk = min(m, 128)
    out = jax.ShapeDtypeStruct((nc, 8, m), jnp.float32)
    nxc, nyc, nzc = pl.pallas_call(
        functools.partial(_fps_body, m, chunk, rpb),
        out_shape=(out, out, out),
    )(X.reshape(nc, 8, w), Y.reshape(nc, 8, w), Z.reshape(nc, 8, w))
    nx = nxc[:, ::rpb, :].reshape(b, m)
    ny = nyc[:, ::rpb, :].reshape(b, m)
    nz = nzc[:, ::rpb, :].reshape(b, m)
    return nx, ny, nz


def _knn_body(n, nk, ng, x_ref, y_ref, z_ref, qx_ref, qy_ref, qz_ref,
              out_ref):
    bi = pl.program_id(0)
    X = x_ref[0]
    Y = y_ref[0]
    Z = z_ref[0]
    bf = jnp.bfloat16
    f32 = jnp.float32
    Xb = X.astype(bf).astype(f32)
    Yb = Y.astype(bf).astype(f32)
    Zb = Z.astype(bf).astype(f32)
    nb = X * X + Y * Y + Z * Z
    iota = jax.lax.broadcasted_iota(jnp.int32, (8, n), 1)
    kl = jax.lax.broadcasted_iota(jnp.int32, (8, nk), 1)
    accs = []
    for g in range(ng):
        qx = qx_ref[g]
        qy = qy_ref[g]
        qz = qz_ref[g]
        mm = (qx.astype(bf).astype(f32) * Xb
              + qy.astype(bf).astype(f32) * Yb
              + qz.astype(bf).astype(f32) * Zb)
        na = qx * qx + qy * qy + qz * qz
        D = (na + nb) - 2.0 * mm
        acc = jnp.zeros((8, nk), jnp.int32)
        for j in range(nk):
            am = jnp.argmin(D, axis=1, keepdims=True).astype(jnp.int32)
            acc = jnp.where(kl == j, am, acc)
            D = jnp.where(iota == am, jnp.inf, D)
        accs.append(acc)
    out_ref[0] = jnp.concatenate(accs, axis=0) + bi * n


def _knn(X, Y, Z, qx, qy, qz, nk, ng=8):
    b, n = X.shape
    s = qx.shape[1]
    grid = (b, s // (8 * ng))
    X3 = X.reshape(b, 1, n)
    Y3 = Y.reshape(b, 1, n)
    Z3 = Z.reshape(b, 1, n)
    qx3 = qx.reshape(b * s // 8, 8, 1)
    qy3 = qy.reshape(b * s // 8, 8, 1)
    qz3 = qz.reshape(b * s // 8, 8, 1)
    nblk = s // (8 * ng)
    pts_spec = pl.BlockSpec((1, 1, n), lambda bi, qi: (bi, 0, 0))
    q_spec = pl.BlockSpec((ng, 8, 1), lambda bi, qi: (bi * nblk + qi, 0, 0))
    return pl.pallas_call(
        functools.partial(_knn_body, n, nk, ng),
        grid=grid,
        in_specs=[pts_spec] * 3 + [q_spec] * 3,
        out_specs=pl.BlockSpec((1, 8 * ng, nk), lambda bi, qi: (bi, qi, 0)),
        out_shape=jax.ShapeDtypeStruct((b, s, nk), jnp.int32),
        compiler_params=pltpu.CompilerParams(
            dimension_semantics=("parallel", "parallel")),
    )(X3, Y3, Z3, qx3, qy3, qz3)


def _pre1_body(xyz_ref, wproj_ref, bproj_ref, w10_ref, b10_ref, out_ref):
    w10x = w10_ref[0:3, :]
    w10f = w10_ref[3:, :]
    M = w10x + jnp.dot(wproj_ref[...], w10f,
                       preferred_element_type=jnp.float32)
    c = jnp.dot(bproj_ref[...], w10f,
                preferred_element_type=jnp.float32) + b10_ref[...]
    x = xyz_ref[0, :, 0:1]
    y = xyz_ref[0, :, 1:2]
    z = xyz_ref[0, :, 2:3]
    out_ref[0] = x * M[0:1, :] + y * M[1:2, :] + z * M[2:3, :] + c


def _pre1(xyz, W_proj, b_proj2, W10, b10_2):
    b, n, _ = xyz.shape
    C = W10.shape[1]
    return pl.pallas_call(
        _pre1_body,
        grid=(b,),
        in_specs=[
            pl.BlockSpec((1, n, 3), lambda bi: (bi, 0, 0)),
            pl.BlockSpec(W_proj.shape, lambda bi: (0, 0)),
            pl.BlockSpec(b_proj2.shape, lambda bi: (0, 0)),
            pl.BlockSpec(W10.shape, lambda bi: (0, 0)),
            pl.BlockSpec(b10_2.shape, lambda bi: (0, 0)),
        ],
        out_specs=pl.BlockSpec((1, n, C), lambda bi: (bi, 0, 0)),
        out_shape=jax.ShapeDtypeStruct((b, n, C), jnp.float32),
        compiler_params=pltpu.CompilerParams(
            dimension_semantics=("parallel",)),
    )(xyz, W_proj, b_proj2, W10, b10_2)


def _pre2_body(xyz_ref, f_ref, w20_ref, b20_ref, out_ref):
    w20x = w20_ref[0:3, :]
    w20f = w20_ref[3:, :]
    x = xyz_ref[0, :, 0:1]
    y = xyz_ref[0, :, 1:2]
    z = xyz_ref[0, :, 2:3]
    acc = jnp.dot(f_ref[0], w20f, preferred_element_type=jnp.float32)
    out_ref[0] = (acc + x * w20x[0:1, :] + y * w20x[1:2, :]
                  + z * w20x[2:3, :] + b20_ref[...])


def _pre2(xyz1, feats1, W20, b20_2):
    b, s, F = feats1.shape
    C = W20.shape[1]
    return pl.pallas_call(
        _pre2_body,
        grid=(b,),
        in_specs=[
            pl.BlockSpec((1, s, 3), lambda bi: (bi, 0, 0)),
            pl.BlockSpec((1, s, F), lambda bi: (bi, 0, 0)),
            pl.BlockSpec(W20.shape, lambda bi: (0, 0)),
            pl.BlockSpec(b20_2.shape, lambda bi: (0, 0)),
        ],
        out_specs=pl.BlockSpec((1, s, C), lambda bi: (bi, 0, 0)),
        out_shape=jax.ShapeDtypeStruct((b, s, C), jnp.float32),
        compiler_params=pltpu.CompilerParams(
            dimension_semantics=("parallel",)),
    )(xyz1, feats1, W20, b20_2)


def _sc_gather(table, flat_idx, window=128):
    nidx = flat_idx.shape[1]
    vdim = table.shape[1]
    mesh = plsc.VectorSubcoreMesh(core_axis_name="core",
                                  subcore_axis_name="subcore")

    @pl.kernel(out_type=jax.ShapeDtypeStruct((nidx, vdim), table.dtype),
               mesh=mesh)
    def gather_kernel(x_hbm, i_hbm, o_hbm):
        def body(i_vmem, o_vmem):
            pltpu.sync_copy(x_hbm.at[i_vmem.at[0]], o_vmem)

        pltpu.emit_pipeline(
            body,
            grid=(nidx // window,),
            in_specs=[pl.BlockSpec((1, window), lambda i: (0, i))],
            out_specs=[pl.BlockSpec((window, vdim), lambda i: (i, 0))],
            core_axis_name=("core", "subcore"),
            dimension_semantics=(pltpu.PARALLEL,),
        )(i_hbm, o_hbm)

    return gather_kernel(table, flat_idx)


def _mlp_body(bq, nk, g_ref, q_ref, w0_ref, w1_ref, b1_ref, out_ref):
    C = w1_ref.shape[0]
    w0x = w0_ref[0:3, :]
    q = q_ref[0]
    qoff = (q[:, 0:1] * w0x[0:1, :] + q[:, 1:2] * w0x[1:2, :]
            + q[:, 2:3] * w0x[2:3, :])
    G = g_ref[0]
    h0 = jnp.maximum(G.reshape(bq, nk, C) - qoff[:, None, :], 0.0)
    h1 = jnp.dot(h0.reshape(bq * nk, C), w1_ref[...],
                 preferred_element_type=jnp.float32) + b1_ref[...]
    h1 = jnp.maximum(h1, 0.0)
    out_ref[0] = jnp.max(h1.reshape(bq, nk, C), axis=1)


def _mlp(G, new_xyz, W0, W1, b1_2, nk, bq):
    b, s, _ = new_xyz.shape
    C = W1.shape[0]
    grid = (b, s // bq)
    return pl.pallas_call(
        functools.partial(_mlp_body, bq, nk),
        grid=grid,
        in_specs=[
            pl.BlockSpec((1, bq * nk, C), lambda bi, qi: (bi, qi, 0)),
            pl.BlockSpec((1, bq, 3), lambda bi, qi: (bi, qi, 0)),
            pl.BlockSpec(W0.shape, lambda bi, qi: (0, 0)),
            pl.BlockSpec(W1.shape, lambda bi, qi: (0, 0)),
            pl.BlockSpec(b1_2.shape, lambda bi, qi: (0, 0)),
        ],
        out_specs=pl.BlockSpec((1, bq, C), lambda bi, qi: (bi, qi, 0)),
        out_shape=jax.ShapeDtypeStruct((b, s, C), jnp.float32),
        compiler_params=pltpu.CompilerParams(
            dimension_semantics=("parallel", "parallel")),
    )(G, new_xyz, W0, W1, b1_2)


def _mlp2_body(bq, nk, g_ref, q_ref, w0_ref, w1_ref, b1_ref,
               wd1_ref, bd1_ref, wd2_ref, bd2_ref, out_ref):
    C = w1_ref.shape[0]
    w0x = w0_ref[0:3, :]
    q = q_ref[0]
    qoff = (q[:, 0:1] * w0x[0:1, :] + q[:, 1:2] * w0x[1:2, :]
            + q[:, 2:3] * w0x[2:3, :])
    G = g_ref[0]
    h0 = jnp.maximum(G.reshape(bq, nk, C) - qoff[:, None, :], 0.0)
    h1 = jnp.dot(h0.reshape(bq * nk, C), w1_ref[...],
                 preferred_element_type=jnp.float32) + b1_ref[...]
    h1 = jnp.maximum(h1, 0.0)
    f2 = jnp.max(h1.reshape(bq, nk, C), axis=1)
    fmax = jnp.max(f2, axis=0, keepdims=True)
    favg = jnp.sum(f2, axis=0, keepdims=True) * (1.0 / bq)
    gfeat = jnp.concatenate([fmax, favg], axis=1)
    h = jnp.dot(gfeat, wd1_ref[...],
                preferred_element_type=jnp.float32) + bd1_ref[...]
    h = jnp.maximum(h, 0.0)
    o = jnp.dot(h, wd2_ref[...],
                preferred_element_type=jnp.float32) + bd2_ref[...]
    out_ref[0] = jnp.maximum(o, 0.0)


def _mlp2(G, new_xyz, W0, W1, b1_2, Wd1, bd1_2, Wd2, bd2_2, nk):
    b, s, _ = new_xyz.shape
    C = W1.shape[0]
    O = Wd2.shape[1]
    return pl.pallas_call(
        functools.partial(_mlp2_body, s, nk),
        grid=(b,),
        in_specs=[
            pl.BlockSpec((1, s * nk, C), lambda bi: (bi, 0, 0)),
            pl.BlockSpec((1, s, 3), lambda bi: (bi, 0, 0)),
            pl.BlockSpec(W0.shape, lambda bi: (0, 0)),
            pl.BlockSpec(W1.shape, lambda bi: (0, 0)),
            pl.BlockSpec(b1_2.shape, lambda bi: (0, 0)),
            pl.BlockSpec(Wd1.shape, lambda bi: (0, 0)),
            pl.BlockSpec(bd1_2.shape, lambda bi: (0, 0)),
            pl.BlockSpec(Wd2.shape, lambda bi: (0, 0)),
            pl.BlockSpec(bd2_2.shape, lambda bi: (0, 0)),
        ],
        out_specs=pl.BlockSpec((1, 1, O), lambda bi: (bi, 0, 0)),
        out_shape=jax.ShapeDtypeStruct((b, 1, O), jnp.float32),
        compiler_params=pltpu.CompilerParams(
            dimension_semantics=("parallel",)),
    )(G, new_xyz, W0, W1, b1_2, Wd1, bd1_2, Wd2, bd2_2)


def kernel(xyz, W_proj, b_proj, W10, b10, W11, b11, W20, b20, W21, b21,
           Wd1, bd1, Wd2, bd2):
    b, n, _ = xyz.shape
    m1, m2, nk = 1024, 256, 16
    C1 = W11.shape[0]
    C2 = W21.shape[0]

    X = xyz[..., 0]
    Y = xyz[..., 1]
    Z = xyz[..., 2]

    nx1, ny1, nz1 = _fps(X, Y, Z, m1)
    flat1 = _knn(X, Y, Z, nx1, ny1, nz1, nk)
    preA1 = _pre1(xyz, W_proj, b_proj.reshape(1, -1), W10,
                  b10.reshape(1, -1))
    G1 = _sc_gather(preA1.reshape(b * n, C1), flat1.reshape(1, -1))
    new_xyz1 = jnp.stack([nx1, ny1, nz1], axis=-1)
    feats1 = _mlp(G1.reshape(b, m1 * nk, C1), new_xyz1, W10, W11,
                  b11.reshape(1, -1), nk, bq=128)

    nx2, ny2, nz2 = _fps(nx1, ny1, nz1, m2)
    flat2 = _knn(nx1, ny1, nz1, nx2, ny2, nz2, nk)
    preA2 = _pre2(new_xyz1, feats1, W20, b20.reshape(1, -1))
    G2 = _sc_gather(preA2.reshape(b * m1, C2), flat2.reshape(1, -1))
    new_xyz2 = jnp.stack([nx2, ny2, nz2], axis=-1)

    out = _mlp2(G2.reshape(b, m2 * nk, C2), new_xyz2, W20, W21,
                b21.reshape(1, -1), Wd1, bd1.reshape(1, -1), Wd2,
                bd2.reshape(1, -1), nk)
    return out


# speedup vs baseline: 16.6681x; 1.0217x over previous
import functools

import jax
import jax.numpy as jnp
from jax.experimental import pallas as pl
from jax.experimental.pallas import tpu as pltpu
from jax.experimental.pallas import tpu_sc as plsc


def _fps_body(m, chunk, rpb, x_ref, y_ref, z_ref, nx_ref, ny_ref, nz_ref):
    nc, _, w = x_ref.shape
    rp = jax.lax.broadcasted_iota(jnp.int32, (8, 1), 0) % rpb
    lanec = jax.lax.broadcasted_iota(jnp.int32, (8, chunk), 1)
    lane_w = jax.lax.broadcasted_iota(jnp.int32, (8, w), 1)
    iota_flat = lane_w + rp * w

    def gbcast(v):
        stride = 1
        while stride < rpb:
            vr = pltpu.roll(v, stride, axis=0)
            sel = (rp % (2 * stride)) >= stride
            v = jnp.where(sel, vr, v)
            stride *= 2
        return v

    def gmax(mv, fl):
        stride = 1
        while stride < rpb:
            mv2 = pltpu.roll(mv, 8 - stride, axis=0)
            fl2 = pltpu.roll(fl, 8 - stride, axis=0)
            take = mv2 > mv
            mv = jnp.where(take, mv2, mv)
            fl = jnp.where(take, fl2, fl)
            stride *= 2
        return gbcast(fl)

    def gsum(v):
        stride = 1
        while stride < rpb:
            v = v + pltpu.roll(v, 8 - stride, axis=0)
            stride *= 2
        return gbcast(v)

    X = [x_ref[c] for c in range(nc)]
    Y = [y_ref[c] for c in range(nc)]
    Z = [z_ref[c] for c in range(nc)]
    lx = [gbcast(X[c][:, 0:1]) for c in range(nc)]
    ly = [gbcast(Y[c][:, 0:1]) for c in range(nc)]
    lz = [gbcast(Z[c][:, 0:1]) for c in range(nc)]
    dists = [jnp.full((8, w), 1e10, jnp.float32) for _ in range(nc)]

    def step(j, carry):
        ds, lxs, lys, lzs, axs, ays, azs = [list(t) for t in carry]
        for c in range(nc):
            d = ((X[c] - lxs[c]) ** 2 + (Y[c] - lys[c]) ** 2
                 + (Z[c] - lzs[c]) ** 2)
            dd = jnp.minimum(ds[c], d)
            am = jnp.argmax(dd, axis=1, keepdims=True).astype(jnp.int32)
            mv = jnp.max(dd, axis=1, keepdims=True)
            fl = gmax(mv, am + rp * w)
            oh = iota_flat == fl
            nlx = gsum(jnp.sum(jnp.where(oh, X[c], 0.0), axis=1,
                               keepdims=True))
            nly = gsum(jnp.sum(jnp.where(oh, Y[c], 0.0), axis=1,
                               keepdims=True))
            nlz = gsum(jnp.sum(jnp.where(oh, Z[c], 0.0), axis=1,
                               keepdims=True))
            colm = lanec == j
            ds[c] = dd
            lxs[c], lys[c], lzs[c] = nlx, nly, nlz
            axs[c] = jnp.where(colm, nlx, axs[c])
            ays[c] = jnp.where(colm, nly, ays[c])
            azs[c] = jnp.where(colm, nlz, azs[c])
        return (tuple(ds), tuple(lxs), tuple(lys), tuple(lzs),
                tuple(axs), tuple(ays), tuple(azs))

    for ci in range(m // chunk):
        axs = [jnp.zeros((8, chunk), jnp.float32) for _ in range(nc)]
        ays = [jnp.zeros((8, chunk), jnp.float32) for _ in range(nc)]
        azs = [jnp.zeros((8, chunk), jnp.float32) for _ in range(nc)]
        if ci == 0:
            axs = [jnp.where(lanec == 0, lx[c], axs[c]) for c in range(nc)]
            ays = [jnp.where(lanec == 0, ly[c], ays[c]) for c in range(nc)]
            azs = [jnp.where(lanec == 0, lz[c], azs[c]) for c in range(nc)]
            start = 1
        else:
            start = 0
        carry = (tuple(dists), tuple(lx), tuple(ly), tuple(lz),
                 tuple(axs), tuple(ays), tuple(azs))
        carry = jax.lax.fori_loop(start, chunk, step, carry, unroll=2)
        dists, lx, ly, lz, axs, ays, azs = [list(t) for t in carry]
        sl = pl.ds(ci * chunk, chunk)
        for c in range(nc):
            nx_ref[c, :, sl] = axs[c]
            ny_ref[c, :, sl] = ays[c]
            nz_ref[c, :, sl] = azs[c]


def _fps(X, Y, Z, m, nc=4):
    b, n = X.shape
    rpb = 8 * nc // b
    w = n // rpb
    chunk = min(m, 128)
    out = jax.ShapeDtypeStruct((nc, 8, m), jnp.float32)
    nxc, nyc, nzc = pl.pallas_call(
        functools.partial(_fps_body, m, chunk, rpb),
        out_shape=(out, out, out),
    )(X.reshape(nc, 8, w), Y.reshape(nc, 8, w), Z.reshape(nc, 8, w))
    nx = nxc[:, ::rpb, :].reshape(b, m)
    ny = nyc[:, ::rpb, :].reshape(b, m)
    nz = nzc[:, ::rpb, :].reshape(b, m)
    return nx, ny, nz


def _knn_body(n, nk, ng, x_ref, y_ref, z_ref, qx_ref, qy_ref, qz_ref,
              out_ref):
    bi = pl.program_id(0)
    X = x_ref[0]
    Y = y_ref[0]
    Z = z_ref[0]
    bf = jnp.bfloat16
    f32 = jnp.float32
    Xb = X.astype(bf).astype(f32)
    Yb = Y.astype(bf).astype(f32)
    Zb = Z.astype(bf).astype(f32)
    nb = X * X + Y * Y + Z * Z
    iota = jax.lax.broadcasted_iota(jnp.int32, (8, n), 1)
    kl = jax.lax.broadcasted_iota(jnp.int32, (8, nk), 1)
    accs = []
    for g in range(ng):
        qx = qx_ref[g]
        qy = qy_ref[g]
        qz = qz_ref[g]
        mm = (qx.astype(bf).astype(f32) * Xb
              + qy.astype(bf).astype(f32) * Yb
              + qz.astype(bf).astype(f32) * Zb)
        na = qx * qx + qy * qy + qz * qz
        D = (na + nb) - 2.0 * mm
        acc = jnp.zeros((8, nk), jnp.int32)
        for j in range(nk):
            am = jnp.argmin(D, axis=1, keepdims=True).astype(jnp.int32)
            acc = jnp.where(kl == j, am, acc)
            D = jnp.where(iota == am, jnp.inf, D)
        accs.append(acc)
    out_ref[0] = jnp.concatenate(accs, axis=0) + bi * n


def _knn(X, Y, Z, qx, qy, qz, nk, ng=8):
    b, n = X.shape
    s = qx.shape[1]
    grid = (b, s // (8 * ng))
    X3 = X.reshape(b, 1, n)
    Y3 = Y.reshape(b, 1, n)
    Z3 = Z.reshape(b, 1, n)
    qx3 = qx.reshape(b * s // 8, 8, 1)
    qy3 = qy.reshape(b * s // 8, 8, 1)
    qz3 = qz.reshape(b * s // 8, 8, 1)
    nblk = s // (8 * ng)
    pts_spec = pl.BlockSpec((1, 1, n), lambda bi, qi: (bi, 0, 0))
    q_spec = pl.BlockSpec((ng, 8, 1), lambda bi, qi: (bi * nblk + qi, 0, 0))
    return pl.pallas_call(
        functools.partial(_knn_body, n, nk, ng),
        grid=grid,
        in_specs=[pts_spec] * 3 + [q_spec] * 3,
        out_specs=pl.BlockSpec((1, 8 * ng, nk), lambda bi, qi: (bi, qi, 0)),
        out_shape=jax.ShapeDtypeStruct((b, s, nk), jnp.int32),
        compiler_params=pltpu.CompilerParams(
            dimension_semantics=("parallel", "parallel")),
    )(X3, Y3, Z3, qx3, qy3, qz3)


def _pre1_body(xyz_ref, wproj_ref, bproj_ref, w10_ref, b10_ref, out_ref):
    w10x = w10_ref[0:3, :]
    w10f = w10_ref[3:, :]
    M = w10x + jnp.dot(wproj_ref[...], w10f,
                       preferred_element_type=jnp.float32)
    c = jnp.dot(bproj_ref[...], w10f,
                preferred_element_type=jnp.float32) + b10_ref[...]
    x = xyz_ref[0, :, 0:1]
    y = xyz_ref[0, :, 1:2]
    z = xyz_ref[0, :, 2:3]
    out_ref[0] = x * M[0:1, :] + y * M[1:2, :] + z * M[2:3, :] + c


def _pre1(xyz, W_proj, b_proj2, W10, b10_2):
    b, n, _ = xyz.shape
    C = W10.shape[1]
    return pl.pallas_call(
        _pre1_body,
        grid=(b,),
        in_specs=[
            pl.BlockSpec((1, n, 3), lambda bi: (bi, 0, 0)),
            pl.BlockSpec(W_proj.shape, lambda bi: (0, 0)),
            pl.BlockSpec(b_proj2.shape, lambda bi: (0, 0)),
            pl.BlockSpec(W10.shape, lambda bi: (0, 0)),
            pl.BlockSpec(b10_2.shape, lambda bi: (0, 0)),
        ],
        out_specs=pl.BlockSpec((1, n, C), lambda bi: (bi, 0, 0)),
        out_shape=jax.ShapeDtypeStruct((b, n, C), jnp.float32),
        compiler_params=pltpu.CompilerParams(
            dimension_semantics=("parallel",)),
    )(xyz, W_proj, b_proj2, W10, b10_2)


def _pre2_body(xyz_ref, f_ref, w20_ref, b20_ref, out_ref):
    w20x = w20_ref[0:3, :]
    w20f = w20_ref[3:, :]
    x = xyz_ref[0, :, 0:1]
    y = xyz_ref[0, :, 1:2]
    z = xyz_ref[0, :, 2:3]
    acc = jnp.dot(f_ref[0], w20f, preferred_element_type=jnp.float32)
    out_ref[0] = (acc + x * w20x[0:1, :] + y * w20x[1:2, :]
                  + z * w20x[2:3, :] + b20_ref[...])


def _pre2(xyz1, feats1, W20, b20_2):
    b, s, F = feats1.shape
    C = W20.shape[1]
    return pl.pallas_call(
        _pre2_body,
        grid=(b,),
        in_specs=[
            pl.BlockSpec((1, s, 3), lambda bi: (bi, 0, 0)),
            pl.BlockSpec((1, s, F), lambda bi: (bi, 0, 0)),
            pl.BlockSpec(W20.shape, lambda bi: (0, 0)),
            pl.BlockSpec(b20_2.shape, lambda bi: (0, 0)),
        ],
        out_specs=pl.BlockSpec((1, s, C), lambda bi: (bi, 0, 0)),
        out_shape=jax.ShapeDtypeStruct((b, s, C), jnp.float32),
        compiler_params=pltpu.CompilerParams(
            dimension_semantics=("parallel",)),
    )(xyz1, feats1, W20, b20_2)


def _sc_gather(table, flat_idx, window=128):
    nidx = flat_idx.shape[1]
    vdim = table.shape[1]
    mesh = plsc.VectorSubcoreMesh(core_axis_name="core",
                                  subcore_axis_name="subcore")

    @pl.kernel(out_type=jax.ShapeDtypeStruct((nidx, vdim), table.dtype),
               mesh=mesh)
    def gather_kernel(x_hbm, i_hbm, o_hbm):
        def body(i_vmem, o_vmem):
            pltpu.sync_copy(x_hbm.at[i_vmem.at[0]], o_vmem)

        pltpu.emit_pipeline(
            body,
            grid=(nidx // window,),
            in_specs=[pl.BlockSpec((1, window), lambda i: (0, i))],
            out_specs=[pl.BlockSpec((window, vdim), lambda i: (i, 0))],
            core_axis_name=("core", "subcore"),
            dimension_semantics=(pltpu.PARALLEL,),
        )(i_hbm, o_hbm)

    return gather_kernel(table, flat_idx)


def _mlp_body(bq, nk, g_ref, q_ref, w0_ref, w1_ref, b1_ref, out_ref):
    C = w1_ref.shape[0]
    w0x = w0_ref[0:3, :]
    q = q_ref[0]
    qoff = (q[:, 0:1] * w0x[0:1, :] + q[:, 1:2] * w0x[1:2, :]
            + q[:, 2:3] * w0x[2:3, :])
    G = g_ref[0]
    h0 = jnp.maximum(G.reshape(bq, nk, C) - qoff[:, None, :], 0.0)
    h1 = jnp.dot(h0.reshape(bq * nk, C), w1_ref[...],
                 preferred_element_type=jnp.float32) + b1_ref[...]
    h1 = jnp.maximum(h1, 0.0)
    out_ref[0] = jnp.max(h1.reshape(bq, nk, C), axis=1)


def _mlp(G, new_xyz, W0, W1, b1_2, nk, bq):
    b, s, _ = new_xyz.shape
    C = W1.shape[0]
    grid = (b, s // bq)
    return pl.pallas_call(
        functools.partial(_mlp_body, bq, nk),
        grid=grid,
        in_specs=[
            pl.BlockSpec((1, bq * nk, C), lambda bi, qi: (bi, qi, 0)),
            pl.BlockSpec((1, bq, 3), lambda bi, qi: (bi, qi, 0)),
            pl.BlockSpec(W0.shape, lambda bi, qi: (0, 0)),
            pl.BlockSpec(W1.shape, lambda bi, qi: (0, 0)),
            pl.BlockSpec(b1_2.shape, lambda bi, qi: (0, 0)),
        ],
        out_specs=pl.BlockSpec((1, bq, C), lambda bi, qi: (bi, qi, 0)),
        out_shape=jax.ShapeDtypeStruct((b, s, C), jnp.float32),
        compiler_params=pltpu.CompilerParams(
            dimension_semantics=("parallel", "parallel")),
    )(G, new_xyz, W0, W1, b1_2)


def _mlp2_body(bq, nk, g_ref, q_ref, w0_ref, w1_ref, b1_ref,
               wd1_ref, bd1_ref, wd2_ref, bd2_ref, out_ref):
    C = w1_ref.shape[0]
    w0x = w0_ref[0:3, :]
    q = q_ref[0]
    qoff = (q[:, 0:1] * w0x[0:1, :] + q[:, 1:2] * w0x[1:2, :]
            + q[:, 2:3] * w0x[2:3, :])
    G = g_ref[0]
    h0 = jnp.maximum(G.reshape(bq, nk, C) - qoff[:, None, :], 0.0)
    h1 = jnp.dot(h0.reshape(bq * nk, C), w1_ref[...],
                 preferred_element_type=jnp.float32) + b1_ref[...]
    h1 = jnp.maximum(h1, 0.0)
    f2 = jnp.max(h1.reshape(bq, nk, C), axis=1)
    fmax = jnp.max(f2, axis=0, keepdims=True)
    favg = jnp.sum(f2, axis=0, keepdims=True) * (1.0 / bq)
    gfeat = jnp.concatenate([fmax, favg], axis=1)
    h = jnp.dot(gfeat, wd1_ref[...],
                preferred_element_type=jnp.float32) + bd1_ref[...]
    h = jnp.maximum(h, 0.0)
    o = jnp.dot(h, wd2_ref[...],
                preferred_element_type=jnp.float32) + bd2_ref[...]
    out_ref[0] = jnp.maximum(o, 0.0)


def _mlp2(G, new_xyz, W0, W1, b1_2, Wd1, bd1_2, Wd2, bd2_2, nk):
    b, s, _ = new_xyz.shape
    C = W1.shape[0]
    O = Wd2.shape[1]
    return pl.pallas_call(
        functools.partial(_mlp2_body, s, nk),
        grid=(b,),
        in_specs=[
            pl.BlockSpec((1, s * nk, C), lambda bi: (bi, 0, 0)),
            pl.BlockSpec((1, s, 3), lambda bi: (bi, 0, 0)),
            pl.BlockSpec(W0.shape, lambda bi: (0, 0)),
            pl.BlockSpec(W1.shape, lambda bi: (0, 0)),
            pl.BlockSpec(b1_2.shape, lambda bi: (0, 0)),
            pl.BlockSpec(Wd1.shape, lambda bi: (0, 0)),
            pl.BlockSpec(bd1_2.shape, lambda bi: (0, 0)),
            pl.BlockSpec(Wd2.shape, lambda bi: (0, 0)),
            pl.BlockSpec(bd2_2.shape, lambda bi: (0, 0)),
        ],
        out_specs=pl.BlockSpec((1, 1, O), lambda bi: (bi, 0, 0)),
        out_shape=jax.ShapeDtypeStruct((b, 1, O), jnp.float32),
        compiler_params=pltpu.CompilerParams(
            dimension_semantics=("parallel",)),
    )(G, new_xyz, W0, W1, b1_2, Wd1, bd1_2, Wd2, bd2_2)


def kernel(xyz, W_proj, b_proj, W10, b10, W11, b11, W20, b20, W21, b21,
           Wd1, bd1, Wd2, bd2):
    b, n, _ = xyz.shape
    m1, m2, nk = 1024, 256, 16
    C1 = W11.shape[0]
    C2 = W21.shape[0]

    X = xyz[..., 0]
    Y = xyz[..., 1]
    Z = xyz[..., 2]

    nx1, ny1, nz1 = _fps(X, Y, Z, m1)
    flat1 = _knn(X, Y, Z, nx1, ny1, nz1, nk)
    preA1 = _pre1(xyz, W_proj, b_proj.reshape(1, -1), W10,
                  b10.reshape(1, -1))
    G1 = _sc_gather(preA1.reshape(b * n, C1), flat1.reshape(1, -1))
    new_xyz1 = jnp.stack([nx1, ny1, nz1], axis=-1)
    feats1 = _mlp(G1.reshape(b, m1 * nk, C1), new_xyz1, W10, W11,
                  b11.reshape(1, -1), nk, bq=128)

    nx2, ny2, nz2 = _fps(nx1, ny1, nz1, m2)
    flat2 = _knn(nx1, ny1, nz1, nx2, ny2, nz2, nk)
    preA2 = _pre2(new_xyz1, feats1, W20, b20.reshape(1, -1))
    G2 = _sc_gather(preA2.reshape(b * m1, C2), flat2.reshape(1, -1))
    new_xyz2 = jnp.stack([nx2, ny2, nz2], axis=-1)

    out = _mlp2(G2.reshape(b, m2 * nk, C2), new_xyz2, W20, W21,
                b21.reshape(1, -1), Wd1, bd1.reshape(1, -1), Wd2,
                bd2.reshape(1, -1), nk)
    return out


# speedup vs baseline: 17.1286x; 1.0276x over previous
import functools

import jax
import jax.numpy as jnp
from jax.experimental import pallas as pl
from jax.experimental.pallas import tpu as pltpu
from jax.experimental.pallas import tpu_sc as plsc


def _fps_body(m, chunk, rpb, x_ref, y_ref, z_ref, nx_ref, ny_ref, nz_ref):
    nc, _, w = x_ref.shape
    rp = jax.lax.broadcasted_iota(jnp.int32, (8, 1), 0) % rpb
    lanec = jax.lax.broadcasted_iota(jnp.int32, (8, chunk), 1)
    lane_w = jax.lax.broadcasted_iota(jnp.int32, (8, w), 1)
    iota_flat = lane_w + rp * w

    def gbcast(v):
        stride = 1
        while stride < rpb:
            vr = pltpu.roll(v, stride, axis=0)
            sel = (rp % (2 * stride)) >= stride
            v = jnp.where(sel, vr, v)
            stride *= 2
        return v

    def gmax(mv, fl):
        stride = 1
        while stride < rpb:
            mv2 = pltpu.roll(mv, 8 - stride, axis=0)
            fl2 = pltpu.roll(fl, 8 - stride, axis=0)
            take = mv2 > mv
            mv = jnp.where(take, mv2, mv)
            fl = jnp.where(take, fl2, fl)
            stride *= 2
        return gbcast(fl)

    def gsum(v):
        stride = 1
        while stride < rpb:
            v = v + pltpu.roll(v, 8 - stride, axis=0)
            stride *= 2
        return gbcast(v)

    X = [x_ref[c] for c in range(nc)]
    Y = [y_ref[c] for c in range(nc)]
    Z = [z_ref[c] for c in range(nc)]
    lx = [gbcast(X[c][:, 0:1]) for c in range(nc)]
    ly = [gbcast(Y[c][:, 0:1]) for c in range(nc)]
    lz = [gbcast(Z[c][:, 0:1]) for c in range(nc)]
    dists = [jnp.full((8, w), 1e10, jnp.float32) for _ in range(nc)]

    def step(j, carry):
        ds, lxs, lys, lzs, axs, ays, azs = [list(t) for t in carry]
        for c in range(nc):
            d = ((X[c] - lxs[c]) ** 2 + (Y[c] - lys[c]) ** 2
                 + (Z[c] - lzs[c]) ** 2)
            dd = jnp.minimum(ds[c], d)
            am = jnp.argmax(dd, axis=1, keepdims=True).astype(jnp.int32)
            mv = jnp.max(dd, axis=1, keepdims=True)
            fl = gmax(mv, am + rp * w)
            oh = iota_flat == fl
            nlx = gsum(jnp.sum(jnp.where(oh, X[c], 0.0), axis=1,
                               keepdims=True))
            nly = gsum(jnp.sum(jnp.where(oh, Y[c], 0.0), axis=1,
                               keepdims=True))
            nlz = gsum(jnp.sum(jnp.where(oh, Z[c], 0.0), axis=1,
                               keepdims=True))
            colm = lanec == j
            ds[c] = dd
            lxs[c], lys[c], lzs[c] = nlx, nly, nlz
            axs[c] = jnp.where(colm, nlx, axs[c])
            ays[c] = jnp.where(colm, nly, ays[c])
            azs[c] = jnp.where(colm, nlz, azs[c])
        return (tuple(ds), tuple(lxs), tuple(lys), tuple(lzs),
                tuple(axs), tuple(ays), tuple(azs))

    for ci in range(m // chunk):
        axs = [jnp.zeros((8, chunk), jnp.float32) for _ in range(nc)]
        ays = [jnp.zeros((8, chunk), jnp.float32) for _ in range(nc)]
        azs = [jnp.zeros((8, chunk), jnp.float32) for _ in range(nc)]
        if ci == 0:
            axs = [jnp.where(lanec == 0, lx[c], axs[c]) for c in range(nc)]
            ays = [jnp.where(lanec == 0, ly[c], ays[c]) for c in range(nc)]
            azs = [jnp.where(lanec == 0, lz[c], azs[c]) for c in range(nc)]
            start = 1
        else:
            start = 0
        carry = (tuple(dists), tuple(lx), tuple(ly), tuple(lz),
                 tuple(axs), tuple(ays), tuple(azs))
        carry = jax.lax.fori_loop(start, chunk, step, carry, unroll=4)
        dists, lx, ly, lz, axs, ays, azs = [list(t) for t in carry]
        sl = pl.ds(ci * chunk, chunk)
        for c in range(nc):
            nx_ref[c, :, sl] = axs[c]
            ny_ref[c, :, sl] = ays[c]
            nz_ref[c, :, sl] = azs[c]


def _fps(X, Y, Z, m, nc=4):
    b, n = X.shape
    rpb = 8 * nc // b
    w = n // rpb
    chunk = min(m, 128)
    out = jax.ShapeDtypeStruct((nc, 8, m), jnp.float32)
    nxc, nyc, nzc = pl.pallas_call(
        functools.partial(_fps_body, m, chunk, rpb),
        out_shape=(out, out, out),
    )(X.reshape(nc, 8, w), Y.reshape(nc, 8, w), Z.reshape(nc, 8, w))
    nx = nxc[:, ::rpb, :].reshape(b, m)
    ny = nyc[:, ::rpb, :].reshape(b, m)
    nz = nzc[:, ::rpb, :].reshape(b, m)
    return nx, ny, nz


def _knn_body(n, nk, ng, x_ref, y_ref, z_ref, qx_ref, qy_ref, qz_ref,
              out_ref):
    bi = pl.program_id(0)
    X = x_ref[0]
    Y = y_ref[0]
    Z = z_ref[0]
    bf = jnp.bfloat16
    f32 = jnp.float32
    Xb = X.astype(bf).astype(f32)
    Yb = Y.astype(bf).astype(f32)
    Zb = Z.astype(bf).astype(f32)
    nb = X * X + Y * Y + Z * Z
    iota = jax.lax.broadcasted_iota(jnp.int32, (8, n), 1)
    kl = jax.lax.broadcasted_iota(jnp.int32, (8, nk), 1)
    accs = []
    for g in range(ng):
        qx = qx_ref[g]
        qy = qy_ref[g]
        qz = qz_ref[g]
        mm = (qx.astype(bf).astype(f32) * Xb
              + qy.astype(bf).astype(f32) * Yb
              + qz.astype(bf).astype(f32) * Zb)
        na = qx * qx + qy * qy + qz * qz
        D = (na + nb) - 2.0 * mm
        acc = jnp.zeros((8, nk), jnp.int32)
        for j in range(nk):
            am = jnp.argmin(D, axis=1, keepdims=True).astype(jnp.int32)
            acc = jnp.where(kl == j, am, acc)
            D = jnp.where(iota == am, jnp.inf, D)
        accs.append(acc)
    out_ref[0] = jnp.concatenate(accs, axis=0) + bi * n


def _knn(X, Y, Z, qx, qy, qz, nk, ng=8):
    b, n = X.shape
    s = qx.shape[1]
    grid = (b, s // (8 * ng))
    X3 = X.reshape(b, 1, n)
    Y3 = Y.reshape(b, 1, n)
    Z3 = Z.reshape(b, 1, n)
    qx3 = qx.reshape(b * s // 8, 8, 1)
    qy3 = qy.reshape(b * s // 8, 8, 1)
    qz3 = qz.reshape(b * s // 8, 8, 1)
    nblk = s // (8 * ng)
    pts_spec = pl.BlockSpec((1, 1, n), lambda bi, qi: (bi, 0, 0))
    q_spec = pl.BlockSpec((ng, 8, 1), lambda bi, qi: (bi * nblk + qi, 0, 0))
    return pl.pallas_call(
        functools.partial(_knn_body, n, nk, ng),
        grid=grid,
        in_specs=[pts_spec] * 3 + [q_spec] * 3,
        out_specs=pl.BlockSpec((1, 8 * ng, nk), lambda bi, qi: (bi, qi, 0)),
        out_shape=jax.ShapeDtypeStruct((b, s, nk), jnp.int32),
        compiler_params=pltpu.CompilerParams(
            dimension_semantics=("parallel", "parallel")),
    )(X3, Y3, Z3, qx3, qy3, qz3)


def _pre1_body(xyz_ref, wproj_ref, bproj_ref, w10_ref, b10_ref, out_ref):
    w10x = w10_ref[0:3, :]
    w10f = w10_ref[3:, :]
    M = w10x + jnp.dot(wproj_ref[...], w10f,
                       preferred_element_type=jnp.float32)
    c = jnp.dot(bproj_ref[...], w10f,
                preferred_element_type=jnp.float32) + b10_ref[...]
    x = xyz_ref[0, :, 0:1]
    y = xyz_ref[0, :, 1:2]
    z = xyz_ref[0, :, 2:3]
    out_ref[0] = x * M[0:1, :] + y * M[1:2, :] + z * M[2:3, :] + c


def _pre1(xyz, W_proj, b_proj2, W10, b10_2):
    b, n, _ = xyz.shape
    C = W10.shape[1]
    return pl.pallas_call(
        _pre1_body,
        grid=(b,),
        in_specs=[
            pl.BlockSpec((1, n, 3), lambda bi: (bi, 0, 0)),
            pl.BlockSpec(W_proj.shape, lambda bi: (0, 0)),
            pl.BlockSpec(b_proj2.shape, lambda bi: (0, 0)),
            pl.BlockSpec(W10.shape, lambda bi: (0, 0)),
            pl.BlockSpec(b10_2.shape, lambda bi: (0, 0)),
        ],
        out_specs=pl.BlockSpec((1, n, C), lambda bi: (bi, 0, 0)),
        out_shape=jax.ShapeDtypeStruct((b, n, C), jnp.float32),
        compiler_params=pltpu.CompilerParams(
            dimension_semantics=("parallel",)),
    )(xyz, W_proj, b_proj2, W10, b10_2)


def _pre2_body(xyz_ref, f_ref, w20_ref, b20_ref, out_ref):
    w20x = w20_ref[0:3, :]
    w20f = w20_ref[3:, :]
    x = xyz_ref[0, :, 0:1]
    y = xyz_ref[0, :, 1:2]
    z = xyz_ref[0, :, 2:3]
    acc = jnp.dot(f_ref[0], w20f, preferred_element_type=jnp.float32)
    out_ref[0] = (acc + x * w20x[0:1, :] + y * w20x[1:2, :]
                  + z * w20x[2:3, :] + b20_ref[...])


def _pre2(xyz1, feats1, W20, b20_2):
    b, s, F = feats1.shape
    C = W20.shape[1]
    return pl.pallas_call(
        _pre2_body,
        grid=(b,),
        in_specs=[
            pl.BlockSpec((1, s, 3), lambda bi: (bi, 0, 0)),
            pl.BlockSpec((1, s, F), lambda bi: (bi, 0, 0)),
            pl.BlockSpec(W20.shape, lambda bi: (0, 0)),
            pl.BlockSpec(b20_2.shape, lambda bi: (0, 0)),
        ],
        out_specs=pl.BlockSpec((1, s, C), lambda bi: (bi, 0, 0)),
        out_shape=jax.ShapeDtypeStruct((b, s, C), jnp.float32),
        compiler_params=pltpu.CompilerParams(
            dimension_semantics=("parallel",)),
    )(xyz1, feats1, W20, b20_2)


def _sc_gather(table, flat_idx, window=128):
    nidx = flat_idx.shape[1]
    vdim = table.shape[1]
    mesh = plsc.VectorSubcoreMesh(core_axis_name="core",
                                  subcore_axis_name="subcore")

    @pl.kernel(out_type=jax.ShapeDtypeStruct((nidx, vdim), table.dtype),
               mesh=mesh)
    def gather_kernel(x_hbm, i_hbm, o_hbm):
        def body(i_vmem, o_vmem):
            pltpu.sync_copy(x_hbm.at[i_vmem.at[0]], o_vmem)

        pltpu.emit_pipeline(
            body,
            grid=(nidx // window,),
            in_specs=[pl.BlockSpec((1, window), lambda i: (0, i))],
            out_specs=[pl.BlockSpec((window, vdim), lambda i: (i, 0))],
            core_axis_name=("core", "subcore"),
            dimension_semantics=(pltpu.PARALLEL,),
        )(i_hbm, o_hbm)

    return gather_kernel(table, flat_idx)


def _mlp_body(bq, nk, g_ref, q_ref, w0_ref, w1_ref, b1_ref, out_ref):
    C = w1_ref.shape[0]
    w0x = w0_ref[0:3, :]
    q = q_ref[0]
    qoff = (q[:, 0:1] * w0x[0:1, :] + q[:, 1:2] * w0x[1:2, :]
            + q[:, 2:3] * w0x[2:3, :])
    G = g_ref[0]
    h0 = jnp.maximum(G.reshape(bq, nk, C) - qoff[:, None, :], 0.0)
    h1 = jnp.dot(h0.reshape(bq * nk, C), w1_ref[...],
                 preferred_element_type=jnp.float32) + b1_ref[...]
    h1 = jnp.maximum(h1, 0.0)
    out_ref[0] = jnp.max(h1.reshape(bq, nk, C), axis=1)


def _mlp(G, new_xyz, W0, W1, b1_2, nk, bq):
    b, s, _ = new_xyz.shape
    C = W1.shape[0]
    grid = (b, s // bq)
    return pl.pallas_call(
        functools.partial(_mlp_body, bq, nk),
        grid=grid,
        in_specs=[
            pl.BlockSpec((1, bq * nk, C), lambda bi, qi: (bi, qi, 0)),
            pl.BlockSpec((1, bq, 3), lambda bi, qi: (bi, qi, 0)),
            pl.BlockSpec(W0.shape, lambda bi, qi: (0, 0)),
            pl.BlockSpec(W1.shape, lambda bi, qi: (0, 0)),
            pl.BlockSpec(b1_2.shape, lambda bi, qi: (0, 0)),
        ],
        out_specs=pl.BlockSpec((1, bq, C), lambda bi, qi: (bi, qi, 0)),
        out_shape=jax.ShapeDtypeStruct((b, s, C), jnp.float32),
        compiler_params=pltpu.CompilerParams(
            dimension_semantics=("parallel", "parallel")),
    )(G, new_xyz, W0, W1, b1_2)


def _mlp2_body(bq, nk, g_ref, q_ref, w0_ref, w1_ref, b1_ref,
               wd1_ref, bd1_ref, wd2_ref, bd2_ref, out_ref):
    C = w1_ref.shape[0]
    w0x = w0_ref[0:3, :]
    q = q_ref[0]
    qoff = (q[:, 0:1] * w0x[0:1, :] + q[:, 1:2] * w0x[1:2, :]
            + q[:, 2:3] * w0x[2:3, :])
    G = g_ref[0]
    h0 = jnp.maximum(G.reshape(bq, nk, C) - qoff[:, None, :], 0.0)
    h1 = jnp.dot(h0.reshape(bq * nk, C), w1_ref[...],
                 preferred_element_type=jnp.float32) + b1_ref[...]
    h1 = jnp.maximum(h1, 0.0)
    f2 = jnp.max(h1.reshape(bq, nk, C), axis=1)
    fmax = jnp.max(f2, axis=0, keepdims=True)
    favg = jnp.sum(f2, axis=0, keepdims=True) * (1.0 / bq)
    gfeat = jnp.concatenate([fmax, favg], axis=1)
    h = jnp.dot(gfeat, wd1_ref[...],
                preferred_element_type=jnp.float32) + bd1_ref[...]
    h = jnp.maximum(h, 0.0)
    o = jnp.dot(h, wd2_ref[...],
                preferred_element_type=jnp.float32) + bd2_ref[...]
    out_ref[0] = jnp.maximum(o, 0.0)


def _mlp2(G, new_xyz, W0, W1, b1_2, Wd1, bd1_2, Wd2, bd2_2, nk):
    b, s, _ = new_xyz.shape
    C = W1.shape[0]
    O = Wd2.shape[1]
    return pl.pallas_call(
        functools.partial(_mlp2_body, s, nk),
        grid=(b,),
        in_specs=[
            pl.BlockSpec((1, s * nk, C), lambda bi: (bi, 0, 0)),
            pl.BlockSpec((1, s, 3), lambda bi: (bi, 0, 0)),
            pl.BlockSpec(W0.shape, lambda bi: (0, 0)),
            pl.BlockSpec(W1.shape, lambda bi: (0, 0)),
            pl.BlockSpec(b1_2.shape, lambda bi: (0, 0)),
            pl.BlockSpec(Wd1.shape, lambda bi: (0, 0)),
            pl.BlockSpec(bd1_2.shape, lambda bi: (0, 0)),
            pl.BlockSpec(Wd2.shape, lambda bi: (0, 0)),
            pl.BlockSpec(bd2_2.shape, lambda bi: (0, 0)),
        ],
        out_specs=pl.BlockSpec((1, 1, O), lambda bi: (bi, 0, 0)),
        out_shape=jax.ShapeDtypeStruct((b, 1, O), jnp.float32),
        compiler_params=pltpu.CompilerParams(
            dimension_semantics=("parallel",)),
    )(G, new_xyz, W0, W1, b1_2, Wd1, bd1_2, Wd2, bd2_2)


def kernel(xyz, W_proj, b_proj, W10, b10, W11, b11, W20, b20, W21, b21,
           Wd1, bd1, Wd2, bd2):
    b, n, _ = xyz.shape
    m1, m2, nk = 1024, 256, 16
    C1 = W11.shape[0]
    C2 = W21.shape[0]

    X = xyz[..., 0]
    Y = xyz[..., 1]
    Z = xyz[..., 2]

    nx1, ny1, nz1 = _fps(X, Y, Z, m1)
    flat1 = _knn(X, Y, Z, nx1, ny1, nz1, nk)
    preA1 = _pre1(xyz, W_proj, b_proj.reshape(1, -1), W10,
                  b10.reshape(1, -1))
    G1 = _sc_gather(preA1.reshape(b * n, C1), flat1.reshape(1, -1))
    new_xyz1 = jnp.stack([nx1, ny1, nz1], axis=-1)
    feats1 = _mlp(G1.reshape(b, m1 * nk, C1), new_xyz1, W10, W11,
                  b11.reshape(1, -1), nk, bq=128)

    nx2, ny2, nz2 = _fps(nx1, ny1, nz1, m2)
    flat2 = _knn(nx1, ny1, nz1, nx2, ny2, nz2, nk)
    preA2 = _pre2(new_xyz1, feats1, W20, b20.reshape(1, -1))
    G2 = _sc_gather(preA2.reshape(b * m1, C2), flat2.reshape(1, -1))
    new_xyz2 = jnp.stack([nx2, ny2, nz2], axis=-1)

    out = _mlp2(G2.reshape(b, m2 * nk, C2), new_xyz2, W20, W21,
                b21.reshape(1, -1), Wd1, bd1.reshape(1, -1), Wd2,
                bd2.reshape(1, -1), nk)
    return out


# speedup vs baseline: 17.2301x; 1.0059x over previous
import functools

import jax
import jax.numpy as jnp
from jax.experimental import pallas as pl
from jax.experimental.pallas import tpu as pltpu
from jax.experimental.pallas import tpu_sc as plsc


def _fps_body(m, chunk, rpb, x_ref, y_ref, z_ref, nx_ref, ny_ref, nz_ref):
    nc, _, w = x_ref.shape
    rp = jax.lax.broadcasted_iota(jnp.int32, (8, 1), 0) % rpb
    lanec = jax.lax.broadcasted_iota(jnp.int32, (8, chunk), 1)
    lane_w = jax.lax.broadcasted_iota(jnp.int32, (8, w), 1)
    iota_flat = lane_w + rp * w

    def gbcast(v):
        stride = 1
        while stride < rpb:
            vr = pltpu.roll(v, stride, axis=0)
            sel = (rp % (2 * stride)) >= stride
            v = jnp.where(sel, vr, v)
            stride *= 2
        return v

    def gmax(mv, fl):
        stride = 1
        while stride < rpb:
            mv2 = pltpu.roll(mv, 8 - stride, axis=0)
            fl2 = pltpu.roll(fl, 8 - stride, axis=0)
            take = mv2 > mv
            mv = jnp.where(take, mv2, mv)
            fl = jnp.where(take, fl2, fl)
            stride *= 2
        return gbcast(fl)

    def gsum(v):
        stride = 1
        while stride < rpb:
            v = v + pltpu.roll(v, 8 - stride, axis=0)
            stride *= 2
        return gbcast(v)

    X = [x_ref[c] for c in range(nc)]
    Y = [y_ref[c] for c in range(nc)]
    Z = [z_ref[c] for c in range(nc)]
    lx = [gbcast(X[c][:, 0:1]) for c in range(nc)]
    ly = [gbcast(Y[c][:, 0:1]) for c in range(nc)]
    lz = [gbcast(Z[c][:, 0:1]) for c in range(nc)]
    dists = [jnp.full((8, w), 1e10, jnp.float32) for _ in range(nc)]

    def step(j, carry):
        ds, lxs, lys, lzs, axs, ays, azs = [list(t) for t in carry]
        for c in range(nc):
            d = ((X[c] - lxs[c]) ** 2 + (Y[c] - lys[c]) ** 2
                 + (Z[c] - lzs[c]) ** 2)
            dd = jnp.minimum(ds[c], d)
            am = jnp.argmax(dd, axis=1, keepdims=True).astype(jnp.int32)
            mv = jnp.max(dd, axis=1, keepdims=True)
            fl = gmax(mv, am + rp * w)
            oh = iota_flat == fl
            nlx = gsum(jnp.sum(jnp.where(oh, X[c], 0.0), axis=1,
                               keepdims=True))
            nly = gsum(jnp.sum(jnp.where(oh, Y[c], 0.0), axis=1,
                               keepdims=True))
            nlz = gsum(jnp.sum(jnp.where(oh, Z[c], 0.0), axis=1,
                               keepdims=True))
            colm = lanec == j
            ds[c] = dd
            lxs[c], lys[c], lzs[c] = nlx, nly, nlz
            axs[c] = jnp.where(colm, nlx, axs[c])
            ays[c] = jnp.where(colm, nly, ays[c])
            azs[c] = jnp.where(colm, nlz, azs[c])
        return (tuple(ds), tuple(lxs), tuple(lys), tuple(lzs),
                tuple(axs), tuple(ays), tuple(azs))

    for ci in range(m // chunk):
        axs = [jnp.zeros((8, chunk), jnp.float32) for _ in range(nc)]
        ays = [jnp.zeros((8, chunk), jnp.float32) for _ in range(nc)]
        azs = [jnp.zeros((8, chunk), jnp.float32) for _ in range(nc)]
        if ci == 0:
            axs = [jnp.where(lanec == 0, lx[c], axs[c]) for c in range(nc)]
            ays = [jnp.where(lanec == 0, ly[c], ays[c]) for c in range(nc)]
            azs = [jnp.where(lanec == 0, lz[c], azs[c]) for c in range(nc)]
            start = 1
        else:
            start = 0
        carry = (tuple(dists), tuple(lx), tuple(ly), tuple(lz),
                 tuple(axs), tuple(ays), tuple(azs))
        carry = jax.lax.fori_loop(start, chunk, step, carry, unroll=8)
        dists, lx, ly, lz, axs, ays, azs = [list(t) for t in carry]
        sl = pl.ds(ci * chunk, chunk)
        for c in range(nc):
            nx_ref[c, :, sl] = axs[c]
            ny_ref[c, :, sl] = ays[c]
            nz_ref[c, :, sl] = azs[c]


def _fps(X, Y, Z, m, nc=4):
    b, n = X.shape
    rpb = 8 * nc // b
    w = n // rpb
    chunk = min(m, 128)
    out = jax.ShapeDtypeStruct((nc, 8, m), jnp.float32)
    nxc, nyc, nzc = pl.pallas_call(
        functools.partial(_fps_body, m, chunk, rpb),
        out_shape=(out, out, out),
    )(X.reshape(nc, 8, w), Y.reshape(nc, 8, w), Z.reshape(nc, 8, w))
    nx = nxc[:, ::rpb, :].reshape(b, m)
    ny = nyc[:, ::rpb, :].reshape(b, m)
    nz = nzc[:, ::rpb, :].reshape(b, m)
    return nx, ny, nz


def _knn_body(n, nk, ng, x_ref, y_ref, z_ref, qx_ref, qy_ref, qz_ref,
              out_ref):
    bi = pl.program_id(0)
    X = x_ref[0]
    Y = y_ref[0]
    Z = z_ref[0]
    bf = jnp.bfloat16
    f32 = jnp.float32
    Xb = X.astype(bf).astype(f32)
    Yb = Y.astype(bf).astype(f32)
    Zb = Z.astype(bf).astype(f32)
    nb = X * X + Y * Y + Z * Z
    iota = jax.lax.broadcasted_iota(jnp.int32, (8, n), 1)
    kl = jax.lax.broadcasted_iota(jnp.int32, (8, nk), 1)
    accs = []
    for g in range(ng):
        qx = qx_ref[g]
        qy = qy_ref[g]
        qz = qz_ref[g]
        mm = (qx.astype(bf).astype(f32) * Xb
              + qy.astype(bf).astype(f32) * Yb
              + qz.astype(bf).astype(f32) * Zb)
        na = qx * qx + qy * qy + qz * qz
        D = (na + nb) - 2.0 * mm
        acc = jnp.zeros((8, nk), jnp.int32)
        for j in range(nk):
            am = jnp.argmin(D, axis=1, keepdims=True).astype(jnp.int32)
            acc = jnp.where(kl == j, am, acc)
            D = jnp.where(iota == am, jnp.inf, D)
        accs.append(acc)
    out_ref[0] = jnp.concatenate(accs, axis=0) + bi * n


def _knn(X, Y, Z, qx, qy, qz, nk, ng=8):
    b, n = X.shape
    s = qx.shape[1]
    grid = (b, s // (8 * ng))
    X3 = X.reshape(b, 1, n)
    Y3 = Y.reshape(b, 1, n)
    Z3 = Z.reshape(b, 1, n)
    qx3 = qx.reshape(b * s // 8, 8, 1)
    qy3 = qy.reshape(b * s // 8, 8, 1)
    qz3 = qz.reshape(b * s // 8, 8, 1)
    nblk = s // (8 * ng)
    pts_spec = pl.BlockSpec((1, 1, n), lambda bi, qi: (bi, 0, 0))
    q_spec = pl.BlockSpec((ng, 8, 1), lambda bi, qi: (bi * nblk + qi, 0, 0))
    return pl.pallas_call(
        functools.partial(_knn_body, n, nk, ng),
        grid=grid,
        in_specs=[pts_spec] * 3 + [q_spec] * 3,
        out_specs=pl.BlockSpec((1, 8 * ng, nk), lambda bi, qi: (bi, qi, 0)),
        out_shape=jax.ShapeDtypeStruct((b, s, nk), jnp.int32),
        compiler_params=pltpu.CompilerParams(
            dimension_semantics=("parallel", "parallel")),
    )(X3, Y3, Z3, qx3, qy3, qz3)


def _pre1_body(xyz_ref, wproj_ref, bproj_ref, w10_ref, b10_ref, out_ref):
    w10x = w10_ref[0:3, :]
    w10f = w10_ref[3:, :]
    M = w10x + jnp.dot(wproj_ref[...], w10f,
                       preferred_element_type=jnp.float32)
    c = jnp.dot(bproj_ref[...], w10f,
                preferred_element_type=jnp.float32) + b10_ref[...]
    x = xyz_ref[0, :, 0:1]
    y = xyz_ref[0, :, 1:2]
    z = xyz_ref[0, :, 2:3]
    out_ref[0] = x * M[0:1, :] + y * M[1:2, :] + z * M[2:3, :] + c


def _pre1(xyz, W_proj, b_proj2, W10, b10_2):
    b, n, _ = xyz.shape
    C = W10.shape[1]
    return pl.pallas_call(
        _pre1_body,
        grid=(b,),
        in_specs=[
            pl.BlockSpec((1, n, 3), lambda bi: (bi, 0, 0)),
            pl.BlockSpec(W_proj.shape, lambda bi: (0, 0)),
            pl.BlockSpec(b_proj2.shape, lambda bi: (0, 0)),
            pl.BlockSpec(W10.shape, lambda bi: (0, 0)),
            pl.BlockSpec(b10_2.shape, lambda bi: (0, 0)),
        ],
        out_specs=pl.BlockSpec((1, n, C), lambda bi: (bi, 0, 0)),
        out_shape=jax.ShapeDtypeStruct((b, n, C), jnp.float32),
        compiler_params=pltpu.CompilerParams(
            dimension_semantics=("parallel",)),
    )(xyz, W_proj, b_proj2, W10, b10_2)


def _pre2_body(xyz_ref, f_ref, w20_ref, b20_ref, out_ref):
    w20x = w20_ref[0:3, :]
    w20f = w20_ref[3:, :]
    x = xyz_ref[0, :, 0:1]
    y = xyz_ref[0, :, 1:2]
    z = xyz_ref[0, :, 2:3]
    acc = jnp.dot(f_ref[0], w20f, preferred_element_type=jnp.float32)
    out_ref[0] = (acc + x * w20x[0:1, :] + y * w20x[1:2, :]
                  + z * w20x[2:3, :] + b20_ref[...])


def _pre2(xyz1, feats1, W20, b20_2):
    b, s, F = feats1.shape
    C = W20.shape[1]
    return pl.pallas_call(
        _pre2_body,
        grid=(b,),
        in_specs=[
            pl.BlockSpec((1, s, 3), lambda bi: (bi, 0, 0)),
            pl.BlockSpec((1, s, F), lambda bi: (bi, 0, 0)),
            pl.BlockSpec(W20.shape, lambda bi: (0, 0)),
            pl.BlockSpec(b20_2.shape, lambda bi: (0, 0)),
        ],
        out_specs=pl.BlockSpec((1, s, C), lambda bi: (bi, 0, 0)),
        out_shape=jax.ShapeDtypeStruct((b, s, C), jnp.float32),
        compiler_params=pltpu.CompilerParams(
            dimension_semantics=("parallel",)),
    )(xyz1, feats1, W20, b20_2)


def _sc_gather(table, flat_idx, window=128):
    nidx = flat_idx.shape[1]
    vdim = table.shape[1]
    mesh = plsc.VectorSubcoreMesh(core_axis_name="core",
                                  subcore_axis_name="subcore")

    @pl.kernel(out_type=jax.ShapeDtypeStruct((nidx, vdim), table.dtype),
               mesh=mesh)
    def gather_kernel(x_hbm, i_hbm, o_hbm):
        def body(i_vmem, o_vmem):
            pltpu.sync_copy(x_hbm.at[i_vmem.at[0]], o_vmem)

        pltpu.emit_pipeline(
            body,
            grid=(nidx // window,),
            in_specs=[pl.BlockSpec((1, window), lambda i: (0, i))],
            out_specs=[pl.BlockSpec((window, vdim), lambda i: (i, 0))],
            core_axis_name=("core", "subcore"),
            dimension_semantics=(pltpu.PARALLEL,),
        )(i_hbm, o_hbm)

    return gather_kernel(table, flat_idx)


def _mlp_body(bq, nk, g_ref, q_ref, w0_ref, w1_ref, b1_ref, out_ref):
    C = w1_ref.shape[0]
    w0x = w0_ref[0:3, :]
    q = q_ref[0]
    qoff = (q[:, 0:1] * w0x[0:1, :] + q[:, 1:2] * w0x[1:2, :]
            + q[:, 2:3] * w0x[2:3, :])
    G = g_ref[0]
    h0 = jnp.maximum(G.reshape(bq, nk, C) - qoff[:, None, :], 0.0)
    h1 = jnp.dot(h0.reshape(bq * nk, C), w1_ref[...],
                 preferred_element_type=jnp.float32) + b1_ref[...]
    h1 = jnp.maximum(h1, 0.0)
    out_ref[0] = jnp.max(h1.reshape(bq, nk, C), axis=1)


def _mlp(G, new_xyz, W0, W1, b1_2, nk, bq):
    b, s, _ = new_xyz.shape
    C = W1.shape[0]
    grid = (b, s // bq)
    return pl.pallas_call(
        functools.partial(_mlp_body, bq, nk),
        grid=grid,
        in_specs=[
            pl.BlockSpec((1, bq * nk, C), lambda bi, qi: (bi, qi, 0)),
            pl.BlockSpec((1, bq, 3), lambda bi, qi: (bi, qi, 0)),
            pl.BlockSpec(W0.shape, lambda bi, qi: (0, 0)),
            pl.BlockSpec(W1.shape, lambda bi, qi: (0, 0)),
            pl.BlockSpec(b1_2.shape, lambda bi, qi: (0, 0)),
        ],
        out_specs=pl.BlockSpec((1, bq, C), lambda bi, qi: (bi, qi, 0)),
        out_shape=jax.ShapeDtypeStruct((b, s, C), jnp.float32),
        compiler_params=pltpu.CompilerParams(
            dimension_semantics=("parallel", "parallel")),
    )(G, new_xyz, W0, W1, b1_2)


def _mlp2_body(bq, nk, g_ref, q_ref, w0_ref, w1_ref, b1_ref,
               wd1_ref, bd1_ref, wd2_ref, bd2_ref, out_ref):
    C = w1_ref.shape[0]
    w0x = w0_ref[0:3, :]
    q = q_ref[0]
    qoff = (q[:, 0:1] * w0x[0:1, :] + q[:, 1:2] * w0x[1:2, :]
            + q[:, 2:3] * w0x[2:3, :])
    G = g_ref[0]
    h0 = jnp.maximum(G.reshape(bq, nk, C) - qoff[:, None, :], 0.0)
    h1 = jnp.dot(h0.reshape(bq * nk, C), w1_ref[...],
                 preferred_element_type=jnp.float32) + b1_ref[...]
    h1 = jnp.maximum(h1, 0.0)
    f2 = jnp.max(h1.reshape(bq, nk, C), axis=1)
    fmax = jnp.max(f2, axis=0, keepdims=True)
    favg = jnp.sum(f2, axis=0, keepdims=True) * (1.0 / bq)
    gfeat = jnp.concatenate([fmax, favg], axis=1)
    h = jnp.dot(gfeat, wd1_ref[...],
                preferred_element_type=jnp.float32) + bd1_ref[...]
    h = jnp.maximum(h, 0.0)
    o = jnp.dot(h, wd2_ref[...],
                preferred_element_type=jnp.float32) + bd2_ref[...]
    out_ref[0] = jnp.maximum(o, 0.0)


def _mlp2(G, new_xyz, W0, W1, b1_2, Wd1, bd1_2, Wd2, bd2_2, nk):
    b, s, _ = new_xyz.shape
    C = W1.shape[0]
    O = Wd2.shape[1]
    return pl.pallas_call(
        functools.partial(_mlp2_body, s, nk),
        grid=(b,),
        in_specs=[
            pl.BlockSpec((1, s * nk, C), lambda bi: (bi, 0, 0)),
            pl.BlockSpec((1, s, 3), lambda bi: (bi, 0, 0)),
            pl.BlockSpec(W0.shape, lambda bi: (0, 0)),
            pl.BlockSpec(W1.shape, lambda bi: (0, 0)),
            pl.BlockSpec(b1_2.shape, lambda bi: (0, 0)),
            pl.BlockSpec(Wd1.shape, lambda bi: (0, 0)),
            pl.BlockSpec(bd1_2.shape, lambda bi: (0, 0)),
            pl.BlockSpec(Wd2.shape, lambda bi: (0, 0)),
            pl.BlockSpec(bd2_2.shape, lambda bi: (0, 0)),
        ],
        out_specs=pl.BlockSpec((1, 1, O), lambda bi: (bi, 0, 0)),
        out_shape=jax.ShapeDtypeStruct((b, 1, O), jnp.float32),
        compiler_params=pltpu.CompilerParams(
            dimension_semantics=("parallel",)),
    )(G, new_xyz, W0, W1, b1_2, Wd1, bd1_2, Wd2, bd2_2)


def kernel(xyz, W_proj, b_proj, W10, b10, W11, b11, W20, b20, W21, b21,
           Wd1, bd1, Wd2, bd2):
    b, n, _ = xyz.shape
    m1, m2, nk = 1024, 256, 16
    C1 = W11.shape[0]
    C2 = W21.shape[0]

    X = xyz[..., 0]
    Y = xyz[..., 1]
    Z = xyz[..., 2]

    nx1, ny1, nz1 = _fps(X, Y, Z, m1)
    flat1 = _knn(X, Y, Z, nx1, ny1, nz1, nk)
    preA1 = _pre1(xyz, W_proj, b_proj.reshape(1, -1), W10,
                  b10.reshape(1, -1))
    G1 = _sc_gather(preA1.reshape(b * n, C1), flat1.reshape(1, -1))
    new_xyz1 = jnp.stack([nx1, ny1, nz1], axis=-1)
    feats1 = _mlp(G1.reshape(b, m1 * nk, C1), new_xyz1, W10, W11,
                  b11.reshape(1, -1), nk, bq=128)

    nx2, ny2, nz2 = _fps(nx1, ny1, nz1, m2)
    flat2 = _knn(nx1, ny1, nz1, nx2, ny2, nz2, nk)
    preA2 = _pre2(new_xyz1, feats1, W20, b20.reshape(1, -1))
    G2 = _sc_gather(preA2.reshape(b * m1, C2), flat2.reshape(1, -1))
    new_xyz2 = jnp.stack([nx2, ny2, nz2], axis=-1)

    out = _mlp2(G2.reshape(b, m2 * nk, C2), new_xyz2, W20, W21,
                b21.reshape(1, -1), Wd1, bd1.reshape(1, -1), Wd2,
                bd2.reshape(1, -1), nk)
    return out
